# baseline (device time: 80979 ns/iter reference)
import jax
import jax.numpy as jnp
from jax import lax
from jax.experimental import pallas as pl
from jax.experimental.pallas import tpu as pltpu

N_DEV = 4
T = 512
D = 256
E = 16
E_LOC = 4
H = 512


def kernel(x, router_W, route_idx, expert_W):
    def body(x_ref, rw_ref, idx_ref, ew_ref, out_ref,
             wown, xcomm, wcomm, pbuf, sbuf, rbuf,
             x_send, x_recv, w_send, w_recv, p_send, p_recv):
        my = lax.axis_index("i")
        left = (my - 1) % N_DEV
        right = (my + 1) % N_DEV

        barrier = pltpu.get_barrier_semaphore()
        for nbr in (left, right):
            pl.semaphore_signal(barrier, inc=1, device_id=(nbr,),
                                device_id_type=pl.DeviceIdType.MESH)
        pl.semaphore_wait(barrier, 2)

        xs = x_ref[:, :]
        scores = jnp.dot(xs, rw_ref[:, :], preferred_element_type=jnp.float32)
        smax = jnp.max(scores, axis=-1, keepdims=True)
        p = jnp.exp(scores - smax)
        p = p / jnp.sum(p, axis=-1, keepdims=True)
        eids = lax.broadcasted_iota(jnp.int32, (T, E), 1)
        m = (eids == idx_ref[:, 0:1]) | (eids == idx_ref[:, 1:2])
        sel = jnp.where(m, p, 0.0)
        w = sel / jnp.sum(sel, axis=-1, keepdims=True)
        wown[:, :] = w

        for h in range(N_DEV - 1):
            xsrc = x_ref if h == 0 else xcomm.at[h - 1]
            wsrc = wown if h == 0 else wcomm.at[h - 1]
            rx = pltpu.make_async_remote_copy(
                src_ref=xsrc, dst_ref=xcomm.at[h],
                send_sem=x_send.at[h], recv_sem=x_recv.at[h],
                device_id=(right,), device_id_type=pl.DeviceIdType.MESH)
            rw = pltpu.make_async_remote_copy(
                src_ref=wsrc, dst_ref=wcomm.at[h],
                send_sem=w_send.at[h], recv_sem=w_recv.at[h],
                device_id=(right,), device_id_type=pl.DeviceIdType.MESH)
            rx.start()
            rw.start()
            rx.wait()
            rw.wait()

        for h in range(N_DEV):
            xc = xs if h == 0 else xcomm[h - 1, :, :]
            wc = w if h == 0 else wcomm[h - 1, :, :]
            acc = jnp.zeros((T, H), dtype=jnp.float32)
            for l in range(E_LOC):
                ge = my * E_LOC + l
                gate = jnp.sum(jnp.where(eids == ge, wc, 0.0),
                               axis=-1, keepdims=True)
                y = jnp.dot(xc, ew_ref[l, :, :],
                            preferred_element_type=jnp.float32)
                acc = acc + gate * y
            pbuf[h, :, :] = acc

        for s in range(N_DEV - 1):
            if s == 0:
                src = pbuf.at[1]
            else:
                sbuf[s - 1, :, :] = rbuf[s - 1, :, :] + pbuf[s + 1, :, :]
                src = sbuf.at[s - 1]
            rp = pltpu.make_async_remote_copy(
                src_ref=src, dst_ref=rbuf.at[s],
                send_sem=p_send.at[s], recv_sem=p_recv.at[s],
                device_id=(right,), device_id_type=pl.DeviceIdType.MESH)
            rp.start()
            rp.wait()

        out_ref[:, :] = rbuf[N_DEV - 2, :, :] + pbuf[0, :, :]

    return pl.pallas_call(
        body,
        out_shape=jax.ShapeDtypeStruct((T, H), jnp.float32),
        in_specs=[pl.BlockSpec(memory_space=pltpu.VMEM)] * 4,
        out_specs=pl.BlockSpec(memory_space=pltpu.VMEM),
        scratch_shapes=[
            pltpu.VMEM((T, E), jnp.float32),
            pltpu.VMEM((N_DEV - 1, T, D), jnp.float32),
            pltpu.VMEM((N_DEV - 1, T, E), jnp.float32),
            pltpu.VMEM((N_DEV, T, H), jnp.float32),
            pltpu.VMEM((2, T, H), jnp.float32),
            pltpu.VMEM((N_DEV - 1, T, H), jnp.float32),
            pltpu.SemaphoreType.DMA((3,)),
            pltpu.SemaphoreType.DMA((3,)),
            pltpu.SemaphoreType.DMA((3,)),
            pltpu.SemaphoreType.DMA((3,)),
            pltpu.SemaphoreType.DMA((3,)),
            pltpu.SemaphoreType.DMA((3,)),
        ],
        compiler_params=pltpu.CompilerParams(collective_id=0),
    )(x, router_W, route_idx, expert_W)


# device time: 49894 ns/iter; 1.6230x vs baseline; 1.6230x over previous
import jax
import jax.numpy as jnp
from jax import lax
from jax.experimental import pallas as pl
from jax.experimental.pallas import tpu as pltpu

N_DEV = 4
T = 512
D = 256
E = 16
E_LOC = 4
H = 512

_OFFS = (2, 1, 3)


def kernel(x, router_W, route_idx, expert_W):
    def body(x_ref, rw_ref, idx_ref, ew_ref, out_ref,
             wown, xcomm, wcomm, psbuf, prbuf,
             x_send, x_recv, w_send, w_recv, p_send, p_recv):
        my = lax.axis_index("i")

        barrier = pltpu.get_barrier_semaphore()
        for o in _OFFS:
            pl.semaphore_signal(barrier, inc=1, device_id=((my + o) % N_DEV,),
                                device_id_type=pl.DeviceIdType.MESH)
        pl.semaphore_wait(barrier, 3)

        pending = []

        for o in _OFFS:
            r = pltpu.make_async_remote_copy(
                src_ref=x_ref, dst_ref=xcomm.at[3 - o],
                send_sem=x_send.at[3 - o], recv_sem=x_recv.at[3 - o],
                device_id=((my + o) % N_DEV,),
                device_id_type=pl.DeviceIdType.MESH)
            r.start()
            pending.append(r)

        xs = x_ref[:, :]
        scores = jnp.dot(xs, rw_ref[:, :], preferred_element_type=jnp.float32)
        smax = jnp.max(scores, axis=-1, keepdims=True)
        p = jnp.exp(scores - smax)
        p = p / jnp.sum(p, axis=-1, keepdims=True)
        eids = lax.broadcasted_iota(jnp.int32, (T, E), 1)
        m = (eids == idx_ref[:, 0:1]) | (eids == idx_ref[:, 1:2])
        sel = jnp.where(m, p, 0.0)
        w = sel / jnp.sum(sel, axis=-1, keepdims=True)
        wown[:, :] = w

        for o in _OFFS:
            r = pltpu.make_async_remote_copy(
                src_ref=wown, dst_ref=wcomm.at[3 - o],
                send_sem=w_send.at[3 - o], recv_sem=w_recv.at[3 - o],
                device_id=((my + o) % N_DEV,),
                device_id_type=pl.DeviceIdType.MESH)
            r.start()
            pending.append(r)

        def partial_for(xc, wc):
            acc = jnp.zeros((T, H), dtype=jnp.float32)
            for l in range(E_LOC):
                ge = my * E_LOC + l
                gate = jnp.sum(jnp.where(eids == ge, wc, 0.0),
                               axis=-1, keepdims=True)
                y = jnp.dot(xc, ew_ref[l, :, :],
                            preferred_element_type=jnp.float32)
                acc = acc + gate * y
            return acc

        pown = partial_for(xs, w)

        for o in _OFFS:
            j = o - 1
            wait_x = pltpu.make_async_remote_copy(
                src_ref=x_ref, dst_ref=xcomm.at[j],
                send_sem=x_send.at[j], recv_sem=x_recv.at[j],
                device_id=((my + o) % N_DEV,),
                device_id_type=pl.DeviceIdType.MESH)
            wait_w = pltpu.make_async_remote_copy(
                src_ref=wown, dst_ref=wcomm.at[j],
                send_sem=w_send.at[j], recv_sem=w_recv.at[j],
                device_id=((my + o) % N_DEV,),
                device_id_type=pl.DeviceIdType.MESH)
            wait_x.wait_recv()
            wait_w.wait_recv()
            psbuf[j, :, :] = partial_for(xcomm[j, :, :], wcomm[j, :, :])
            r = pltpu.make_async_remote_copy(
                src_ref=psbuf.at[j], dst_ref=prbuf.at[3 - o],
                send_sem=p_send.at[j], recv_sem=p_recv.at[3 - o],
                device_id=((my + o) % N_DEV,),
                device_id_type=pl.DeviceIdType.MESH)
            r.start()
            pending.append(r)

        for j in range(N_DEV - 1):
            wait_p = pltpu.make_async_remote_copy(
                src_ref=psbuf.at[j], dst_ref=prbuf.at[j],
                send_sem=p_send.at[j], recv_sem=p_recv.at[j],
                device_id=(my,), device_id_type=pl.DeviceIdType.MESH)
            wait_p.wait_recv()
        out_ref[:, :] = (pown + prbuf[0, :, :]
                         + prbuf[1, :, :] + prbuf[2, :, :])

        for r in pending:
            r.wait_send()

    return pl.pallas_call(
        body,
        out_shape=jax.ShapeDtypeStruct((T, H), jnp.float32),
        in_specs=[pl.BlockSpec(memory_space=pltpu.VMEM)] * 4,
        out_specs=pl.BlockSpec(memory_space=pltpu.VMEM),
        scratch_shapes=[
            pltpu.VMEM((T, E), jnp.float32),
            pltpu.VMEM((N_DEV - 1, T, D), jnp.float32),
            pltpu.VMEM((N_DEV - 1, T, E), jnp.float32),
            pltpu.VMEM((N_DEV - 1, T, H), jnp.float32),
            pltpu.VMEM((N_DEV - 1, T, H), jnp.float32),
            pltpu.SemaphoreType.DMA((3,)),
            pltpu.SemaphoreType.DMA((3,)),
            pltpu.SemaphoreType.DMA((3,)),
            pltpu.SemaphoreType.DMA((3,)),
            pltpu.SemaphoreType.DMA((3,)),
            pltpu.SemaphoreType.DMA((3,)),
        ],
        compiler_params=pltpu.CompilerParams(collective_id=0),
    )(x, router_W, route_idx, expert_W)


# device time: 12224 ns/iter; 6.6246x vs baseline; 4.0816x over previous
import jax
import jax.numpy as jnp
from jax import lax
from jax.experimental import pallas as pl
from jax.experimental.pallas import tpu as pltpu

N_DEV = 4
T = 512
D = 256
E = 16
E_LOC = 4
H = 512

_OFFS = (2, 1, 3)
_WORK = (1, 3, 2)


def kernel(x, router_W, route_idx, expert_W):
    def body(x_ref, rw_ref, idx_ref, ew_ref, out_ref,
             xbf, ewbf, wown, xcomm, wcomm, psbuf, prbuf,
             x_send, x_recv, w_send, w_recv, p_send, p_recv):
        my = lax.axis_index("i")

        barrier = pltpu.get_barrier_semaphore()
        for o in _OFFS:
            pl.semaphore_signal(barrier, inc=1, device_id=((my + o) % N_DEV,),
                                device_id_type=pl.DeviceIdType.MESH)
        pl.semaphore_wait(barrier, 3)

        pending = []

        xbf[:, :] = x_ref[:, :].astype(jnp.bfloat16)
        for o in _OFFS:
            r = pltpu.make_async_remote_copy(
                src_ref=xbf, dst_ref=xcomm.at[3 - o],
                send_sem=x_send.at[3 - o], recv_sem=x_recv.at[3 - o],
                device_id=((my + o) % N_DEV,),
                device_id_type=pl.DeviceIdType.MESH)
            r.start()
            pending.append(r)

        scores = jnp.dot(x_ref[:, :], rw_ref[:, :],
                         preferred_element_type=jnp.float32)
        smax = jnp.max(scores, axis=-1, keepdims=True)
        p = jnp.exp(scores - smax)
        p = p / jnp.sum(p, axis=-1, keepdims=True)
        eids = lax.broadcasted_iota(jnp.int32, (T, E), 1)
        m = (eids == idx_ref[:, 0:1]) | (eids == idx_ref[:, 1:2])
        sel = jnp.where(m, p, 0.0)
        w = sel / jnp.sum(sel, axis=-1, keepdims=True)
        wown[:, :] = w

        for o in _OFFS:
            r = pltpu.make_async_remote_copy(
                src_ref=wown, dst_ref=wcomm.at[3 - o],
                send_sem=w_send.at[3 - o], recv_sem=w_recv.at[3 - o],
                device_id=((my + o) % N_DEV,),
                device_id_type=pl.DeviceIdType.MESH)
            r.start()
            pending.append(r)

        ewbf[:, :, :] = ew_ref[:, :, :].astype(jnp.bfloat16)

        def partial_for(xc, wc):
            acc = jnp.zeros((T, H), dtype=jnp.float32)
            for l in range(E_LOC):
                ge = my * E_LOC + l
                gate = jnp.sum(jnp.where(eids == ge, wc, 0.0),
                               axis=-1, keepdims=True)
                y = jnp.dot(xc, ewbf[l, :, :],
                            preferred_element_type=jnp.float32)
                acc = acc + gate * y
            return acc

        pown = partial_for(xbf[:, :], w)

        for o in _WORK:
            j = o - 1
            wait_x = pltpu.make_async_remote_copy(
                src_ref=xbf, dst_ref=xcomm.at[j],
                send_sem=x_send.at[j], recv_sem=x_recv.at[j],
                device_id=((my + o) % N_DEV,),
                device_id_type=pl.DeviceIdType.MESH)
            wait_w = pltpu.make_async_remote_copy(
                src_ref=wown, dst_ref=wcomm.at[j],
                send_sem=w_send.at[j], recv_sem=w_recv.at[j],
                device_id=((my + o) % N_DEV,),
                device_id_type=pl.DeviceIdType.MESH)
            wait_x.wait_recv()
            wait_w.wait_recv()
            psbuf[j, :, :] = partial_for(
                xcomm[j, :, :], wcomm[j, :, :]).astype(jnp.bfloat16)
            r = pltpu.make_async_remote_copy(
                src_ref=psbuf.at[j], dst_ref=prbuf.at[3 - o],
                send_sem=p_send.at[j], recv_sem=p_recv.at[3 - o],
                device_id=((my + o) % N_DEV,),
                device_id_type=pl.DeviceIdType.MESH)
            r.start()
            pending.append(r)

        for j in range(N_DEV - 1):
            wait_p = pltpu.make_async_remote_copy(
                src_ref=psbuf.at[j], dst_ref=prbuf.at[j],
                send_sem=p_send.at[j], recv_sem=p_recv.at[j],
                device_id=(my,), device_id_type=pl.DeviceIdType.MESH)
            wait_p.wait_recv()
        out_ref[:, :] = (pown
                         + prbuf[0, :, :].astype(jnp.float32)
                         + prbuf[1, :, :].astype(jnp.float32)
                         + prbuf[2, :, :].astype(jnp.float32))

        for r in pending:
            r.wait_send()

    return pl.pallas_call(
        body,
        out_shape=jax.ShapeDtypeStruct((T, H), jnp.float32),
        in_specs=[pl.BlockSpec(memory_space=pltpu.VMEM)] * 4,
        out_specs=pl.BlockSpec(memory_space=pltpu.VMEM),
        scratch_shapes=[
            pltpu.VMEM((T, D), jnp.bfloat16),
            pltpu.VMEM((E_LOC, D, H), jnp.bfloat16),
            pltpu.VMEM((T, E), jnp.float32),
            pltpu.VMEM((N_DEV - 1, T, D), jnp.bfloat16),
            pltpu.VMEM((N_DEV - 1, T, E), jnp.float32),
            pltpu.VMEM((N_DEV - 1, T, H), jnp.bfloat16),
            pltpu.VMEM((N_DEV - 1, T, H), jnp.bfloat16),
            pltpu.SemaphoreType.DMA((3,)),
            pltpu.SemaphoreType.DMA((3,)),
            pltpu.SemaphoreType.DMA((3,)),
            pltpu.SemaphoreType.DMA((3,)),
            pltpu.SemaphoreType.DMA((3,)),
            pltpu.SemaphoreType.DMA((3,)),
        ],
        compiler_params=pltpu.CompilerParams(collective_id=0),
    )(x, router_W, route_idx, expert_W)
